# Initial kernel scaffold; baseline (speedup 1.0000x reference)
import jax
import jax.numpy as jnp
from jax.experimental import pallas as pl


def kernel(x, edge_index, edge_weight, W1, b1, W2, b2, W3, b3, We, be):
    raise NotImplementedError("write your pallas kernel here")


# speedup vs baseline: 4.1660x; 4.1660x over previous
import functools

import jax
import jax.numpy as jnp
from jax import lax
from jax.experimental import pallas as pl
from jax.experimental.pallas import tpu as pltpu
from jax.experimental.pallas import tpu_sc as plsc

_N = 10000
_E = 320000
_NFEAT = 128
_NH1 = 128
_NH2 = 64
_NCLASS = 16
_NSTRUC = 32

_NC = 2
_NS = 16
_NW = _NC * _NS
_EPT = _E // _NW
_CHUNK = 80
_NCH = _EPT // _CHUNK
_PN = 10240
_RPT = _PN // _NS


def _make_sc_agg(D):
  nvec = D // 16
  mesh = plsc.VectorSubcoreMesh(core_axis_name="c", subcore_axis_name="s")

  @functools.partial(
      pl.kernel,
      out_type=jax.ShapeDtypeStruct((_NC, _PN, D), jnp.float32),
      mesh=mesh,
      compiler_params=pltpu.CompilerParams(use_tc_tiling_on_sc=False),
      scratch_types=[
          pltpu.VMEM((_CHUNK,), jnp.int32),
          pltpu.VMEM((_CHUNK,), jnp.int32),
          pltpu.VMEM((_CHUNK,), jnp.float32),
          pltpu.VMEM((_CHUNK, D), jnp.float32),
          pltpu.VMEM_SHARED((_PN, D), jnp.float32),
          pltpu.SemaphoreType.DMA,
      ],
  )
  def k(sup_hbm, src_hbm, dst_hbm, ew_hbm, out_hbm,
        src_v, dst_v, ew_v, rows_v, agg_sh, sem):
    cid = lax.axis_index("c")
    sid = lax.axis_index("s")
    wid = sid * _NC + cid
    ebase = wid * _EPT
    rbase = sid * _RPT

    def zrow(r, carry):
      for c in range(nvec):
        rows_v[r, pl.ds(c * 16, 16)] = jnp.zeros((16,), jnp.float32)
      return carry
    lax.fori_loop(0, _CHUNK, zrow, 0)
    for j in range(_RPT // _CHUNK):
      pltpu.sync_copy(rows_v, agg_sh.at[pl.ds(rbase + j * _CHUNK, _CHUNK)])
    plsc.subcore_barrier()

    def body(i, carry):
      off = ebase + i * _CHUNK
      pltpu.sync_copy(src_hbm.at[pl.ds(off, _CHUNK)], src_v)
      pltpu.sync_copy(dst_hbm.at[pl.ds(off, _CHUNK)], dst_v)
      pltpu.sync_copy(ew_hbm.at[pl.ds(off, _CHUNK)], ew_v)
      pltpu.async_copy(sup_hbm.at[src_v], rows_v, sem).wait()

      def scale(g, c2):
        ew16 = ew_v[pl.ds(g * 16, 16)]
        for j in range(16):
          w = jnp.broadcast_to(ew16[j], (16,))
          r = g * 16 + j
          for c in range(nvec):
            rows_v[r, pl.ds(c * 16, 16)] = rows_v[r, pl.ds(c * 16, 16)] * w
        return c2
      lax.fori_loop(0, _CHUNK // 16, scale, 0)

      pltpu.sync_copy(rows_v, agg_sh.at[dst_v], add=True)
      return carry
    lax.fori_loop(0, _NCH, body, 0)

    plsc.subcore_barrier()
    pltpu.sync_copy(agg_sh.at[pl.ds(rbase, _RPT)],
                    out_hbm.at[cid, pl.ds(rbase, _RPT)])

  return k


_sc_agg = {D: _make_sc_agg(D) for D in (_NH1, _NH2, _NCLASS)}

_BR = 1000


def _tc_mm(x, W):
  def body(x_ref, w_ref, o_ref):
    o_ref[...] = jnp.dot(x_ref[...], w_ref[...],
                         preferred_element_type=jnp.float32)
  return pl.pallas_call(
      body,
      grid=(_N // _BR,),
      in_specs=[pl.BlockSpec((_BR, x.shape[1]), lambda i: (i, 0)),
                pl.BlockSpec(W.shape, lambda i: (0, 0))],
      out_specs=pl.BlockSpec((_BR, W.shape[1]), lambda i: (i, 0)),
      out_shape=jax.ShapeDtypeStruct((_N, W.shape[1]), jnp.float32),
  )(x, W)


def _tc_relu_mm(p, b, W):
  Din, Dout = W.shape
  def body(p_ref, b_ref, w_ref, o_ref):
    h = jnp.maximum(p_ref[0] + p_ref[1] + b_ref[...], 0.0)
    o_ref[...] = jnp.dot(h, w_ref[...], preferred_element_type=jnp.float32)
  return pl.pallas_call(
      body,
      grid=(_N // _BR,),
      in_specs=[pl.BlockSpec((2, _BR, Din), lambda i: (0, i, 0)),
                pl.BlockSpec((1, Din), lambda i: (0, 0)),
                pl.BlockSpec((Din, Dout), lambda i: (0, 0))],
      out_specs=pl.BlockSpec((_BR, Dout), lambda i: (i, 0)),
      out_shape=jax.ShapeDtypeStruct((_N, Dout), jnp.float32),
  )(p, b.reshape(1, Din), W)


def _tc_layer3(p, b2, W3, We, be):
  def body(p_ref, b2_ref, w3_ref, we_ref, be_ref, o1_ref, o2_ref):
    h = jnp.maximum(p_ref[0] + p_ref[1] + b2_ref[...], 0.0)
    o1_ref[...] = jnp.dot(h, w3_ref[...], preferred_element_type=jnp.float32)
    o2_ref[...] = jnp.dot(h, we_ref[...],
                          preferred_element_type=jnp.float32) + be_ref[...]
  return pl.pallas_call(
      body,
      grid=(_N // _BR,),
      in_specs=[pl.BlockSpec((2, _BR, _NH2), lambda i: (0, i, 0)),
                pl.BlockSpec((1, _NH2), lambda i: (0, 0)),
                pl.BlockSpec((_NH2, _NCLASS), lambda i: (0, 0)),
                pl.BlockSpec((_NH2, _NSTRUC), lambda i: (0, 0)),
                pl.BlockSpec((1, _NSTRUC), lambda i: (0, 0))],
      out_specs=[pl.BlockSpec((_BR, _NCLASS), lambda i: (i, 0)),
                 pl.BlockSpec((_BR, _NSTRUC), lambda i: (i, 0))],
      out_shape=[jax.ShapeDtypeStruct((_N, _NCLASS), jnp.float32),
                 jax.ShapeDtypeStruct((_N, _NSTRUC), jnp.float32)],
  )(p, b2.reshape(1, _NH2), W3, We, be.reshape(1, _NSTRUC))


def _tc_logsoftmax(p, b):
  def body(p_ref, b_ref, o_ref):
    o = p_ref[0] + p_ref[1] + b_ref[...]
    s = o - jnp.max(o, axis=1, keepdims=True)
    o_ref[...] = s - jnp.log(jnp.sum(jnp.exp(s), axis=1, keepdims=True))
  return pl.pallas_call(
      body,
      grid=(_N // _BR,),
      in_specs=[pl.BlockSpec((2, _BR, _NCLASS), lambda i: (0, i, 0)),
                pl.BlockSpec((1, _NCLASS), lambda i: (0, 0))],
      out_specs=pl.BlockSpec((_BR, _NCLASS), lambda i: (i, 0)),
      out_shape=jax.ShapeDtypeStruct((_N, _NCLASS), jnp.float32),
  )(p, b.reshape(1, _NCLASS))


def kernel(x, edge_index, edge_weight, W1, b1, W2, b2, W3, b3, We, be):
  src = edge_index[0]
  dst = edge_index[1]
  s1 = _tc_mm(x, W1)
  p1 = _sc_agg[_NH1](s1, src, dst, edge_weight)
  s2 = _tc_relu_mm(p1, b1, W2)
  p2 = _sc_agg[_NH2](s2, src, dst, edge_weight)
  s3, out2 = _tc_layer3(p2, b2, W3, We, be)
  p3 = _sc_agg[_NCLASS](s3, src, dst, edge_weight)
  out1 = _tc_logsoftmax(p3, b3)
  return out1, out2


# speedup vs baseline: 13.1698x; 3.1613x over previous
import functools

import jax
import jax.numpy as jnp
from jax import lax
from jax.experimental import pallas as pl
from jax.experimental.pallas import tpu as pltpu
from jax.experimental.pallas import tpu_sc as plsc

_N = 10000
_E = 320000
_NFEAT = 128
_NH1 = 128
_NH2 = 64
_NCLASS = 16
_NSTRUC = 32

_NC = 2
_NS = 16
_NW = _NC * _NS
_EPT = _E // _NW
_PN = 10240
_RPT = _PN // _NS


def _chunk_for(D):
  return 40 if D == 128 else 80


def _make_sc_agg(D):
  nvec = D // 16
  chunk = _chunk_for(D)
  nch = _EPT // chunk
  mesh = plsc.VectorSubcoreMesh(core_axis_name="c", subcore_axis_name="s")

  @functools.partial(
      pl.kernel,
      out_type=jax.ShapeDtypeStruct((_NC, _PN, D), jnp.float32),
      mesh=mesh,
      compiler_params=pltpu.CompilerParams(use_tc_tiling_on_sc=False),
      scratch_types=[
          pltpu.VMEM((nch, chunk), jnp.int32),
          pltpu.VMEM((nch, chunk), jnp.int32),
          pltpu.VMEM((chunk,), jnp.float32),
          pltpu.VMEM((chunk,), jnp.float32),
          pltpu.VMEM((chunk, D), jnp.float32),
          pltpu.VMEM((chunk, D), jnp.float32),
          pltpu.VMEM((chunk, D), jnp.float32),
          pltpu.VMEM((chunk, D), jnp.float32),
          pltpu.VMEM_SHARED((_PN, D), jnp.float32),
          pltpu.SemaphoreType.DMA,
          pltpu.SemaphoreType.DMA,
          pltpu.SemaphoreType.DMA,
          pltpu.SemaphoreType.DMA,
          pltpu.SemaphoreType.DMA,
          pltpu.SemaphoreType.DMA,
      ],
  )
  def k(sup_hbm, src_hbm, dst_hbm, ew_hbm, out_hbm,
        src_all, dst_all, ew0, ew1, rg0, rg1, rs0, rs1, agg_sh,
        gs0, gs1, ss0, ss1, es0, es1):
    ew = (ew0, ew1)
    rg = (rg0, rg1)
    rs = (rs0, rs1)
    gs = (gs0, gs1)
    ss = (ss0, ss1)
    es = (es0, es1)
    cid = lax.axis_index("c")
    sid = lax.axis_index("s")
    wid = sid * _NC + cid
    rbase = sid * _RPT

    pltpu.sync_copy(src_hbm.at[wid], src_all)
    pltpu.sync_copy(dst_hbm.at[wid], dst_all)

    def zrow(r, carry):
      for c in range(nvec):
        rg0[r, pl.ds(c * 16, 16)] = jnp.zeros((16,), jnp.float32)
      return carry
    lax.fori_loop(0, chunk, zrow, 0)
    for j in range(_RPT // chunk):
      pltpu.sync_copy(rg0, agg_sh.at[pl.ds(rbase + j * chunk, chunk)])
    plsc.subcore_barrier()

    def issue_ew(j, b):
      pltpu.async_copy(ew_hbm.at[wid, j], ew[b], es[b])

    def wait_ew(j, b):
      pltpu.make_async_copy(ew_hbm.at[wid, j], ew[b], es[b]).wait()

    def issue_gather(j, b):
      pltpu.async_copy(sup_hbm.at[src_all.at[j]], rg[b], gs[b])

    def wait_gather(j, b):
      pltpu.make_async_copy(sup_hbm.at[src_all.at[j]], rg[b], gs[b]).wait()

    def issue_scatter(j, b):
      pltpu.async_copy(rs[b], agg_sh.at[dst_all.at[j]], ss[b], add=True)

    def wait_scatter(j, b):
      pltpu.make_async_copy(rs[b], agg_sh.at[dst_all.at[j]], ss[b]).wait()

    groups = [(g * 16, 0) for g in range(chunk // 16)]
    if chunk % 16:
      groups.append((chunk - 16, 16 - chunk % 16))

    def scale(j, b):
      for base, jj0 in groups:
        ew16 = ew[b][pl.ds(base, 16)]
        for jj in range(jj0, 16):
          w = jnp.broadcast_to(ew16[jj], (16,))
          r = base + jj
          for c in range(nvec):
            rs[b][r, pl.ds(c * 16, 16)] = rg[b][r, pl.ds(c * 16, 16)] * w

    def step(j, b):
      wait_gather(j, b)
      wait_ew(j, b)

      @pl.when(j >= 2)
      def _():
        wait_scatter(j, b)

      scale(j, b)

      @pl.when(j <= nch - 3)
      def _():
        issue_gather(j + 2, b)
        issue_ew(j + 2, b)

      issue_scatter(j, b)

    issue_ew(0, 0)
    issue_ew(1, 1)
    issue_gather(0, 0)
    issue_gather(1, 1)

    def body(g, carry):
      step(2 * g, 0)
      step(2 * g + 1, 1)
      return carry
    lax.fori_loop(0, nch // 2, body, 0)
    if nch % 2:
      step(jnp.int32(nch - 1), 0)

    wait_scatter(jnp.int32(nch - 2), (nch - 2) % 2)
    wait_scatter(jnp.int32(nch - 1), (nch - 1) % 2)

    plsc.subcore_barrier()
    pltpu.sync_copy(agg_sh.at[pl.ds(rbase, _RPT)],
                    out_hbm.at[cid, pl.ds(rbase, _RPT)])

  return k


_sc_agg = {D: _make_sc_agg(D) for D in (_NH1, _NH2, _NCLASS)}

_BR = 1000


def _tc_mm(x, W):
  def body(x_ref, w_ref, o_ref):
    o_ref[...] = jnp.dot(x_ref[...], w_ref[...],
                         preferred_element_type=jnp.float32)
  return pl.pallas_call(
      body,
      grid=(_N // _BR,),
      in_specs=[pl.BlockSpec((_BR, x.shape[1]), lambda i: (i, 0)),
                pl.BlockSpec(W.shape, lambda i: (0, 0))],
      out_specs=pl.BlockSpec((_BR, W.shape[1]), lambda i: (i, 0)),
      out_shape=jax.ShapeDtypeStruct((_N, W.shape[1]), jnp.float32),
  )(x, W)


def _tc_relu_mm(p, b, W):
  Din, Dout = W.shape
  def body(p_ref, b_ref, w_ref, o_ref):
    h = jnp.maximum(p_ref[0] + p_ref[1] + b_ref[...], 0.0)
    o_ref[...] = jnp.dot(h, w_ref[...], preferred_element_type=jnp.float32)
  return pl.pallas_call(
      body,
      grid=(_N // _BR,),
      in_specs=[pl.BlockSpec((2, _BR, Din), lambda i: (0, i, 0)),
                pl.BlockSpec((1, Din), lambda i: (0, 0)),
                pl.BlockSpec((Din, Dout), lambda i: (0, 0))],
      out_specs=pl.BlockSpec((_BR, Dout), lambda i: (i, 0)),
      out_shape=jax.ShapeDtypeStruct((_N, Dout), jnp.float32),
  )(p, b.reshape(1, Din), W)


def _tc_layer3(p, b2, W3, We, be):
  def body(p_ref, b2_ref, w3_ref, we_ref, be_ref, o1_ref, o2_ref):
    h = jnp.maximum(p_ref[0] + p_ref[1] + b2_ref[...], 0.0)
    o1_ref[...] = jnp.dot(h, w3_ref[...], preferred_element_type=jnp.float32)
    o2_ref[...] = jnp.dot(h, we_ref[...],
                          preferred_element_type=jnp.float32) + be_ref[...]
  return pl.pallas_call(
      body,
      grid=(_N // _BR,),
      in_specs=[pl.BlockSpec((2, _BR, _NH2), lambda i: (0, i, 0)),
                pl.BlockSpec((1, _NH2), lambda i: (0, 0)),
                pl.BlockSpec((_NH2, _NCLASS), lambda i: (0, 0)),
                pl.BlockSpec((_NH2, _NSTRUC), lambda i: (0, 0)),
                pl.BlockSpec((1, _NSTRUC), lambda i: (0, 0))],
      out_specs=[pl.BlockSpec((_BR, _NCLASS), lambda i: (i, 0)),
                 pl.BlockSpec((_BR, _NSTRUC), lambda i: (i, 0))],
      out_shape=[jax.ShapeDtypeStruct((_N, _NCLASS), jnp.float32),
                 jax.ShapeDtypeStruct((_N, _NSTRUC), jnp.float32)],
  )(p, b2.reshape(1, _NH2), W3, We, be.reshape(1, _NSTRUC))


def _tc_logsoftmax(p, b):
  def body(p_ref, b_ref, o_ref):
    o = p_ref[0] + p_ref[1] + b_ref[...]
    s = o - jnp.max(o, axis=1, keepdims=True)
    o_ref[...] = s - jnp.log(jnp.sum(jnp.exp(s), axis=1, keepdims=True))
  return pl.pallas_call(
      body,
      grid=(_N // _BR,),
      in_specs=[pl.BlockSpec((2, _BR, _NCLASS), lambda i: (0, i, 0)),
                pl.BlockSpec((1, _NCLASS), lambda i: (0, 0))],
      out_specs=pl.BlockSpec((_BR, _NCLASS), lambda i: (i, 0)),
      out_shape=jax.ShapeDtypeStruct((_N, _NCLASS), jnp.float32),
  )(p, b.reshape(1, _NCLASS))


def _edges_for(edge_index, edge_weight, D):
  chunk = _chunk_for(D)
  nch = _EPT // chunk
  return (edge_index[0].reshape(_NW, nch, chunk),
          edge_index[1].reshape(_NW, nch, chunk),
          edge_weight.reshape(_NW, nch, chunk))


def kernel(x, edge_index, edge_weight, W1, b1, W2, b2, W3, b3, We, be):
  src40, dst40, ew40 = _edges_for(edge_index, edge_weight, _NH1)
  src80, dst80, ew80 = _edges_for(edge_index, edge_weight, _NH2)
  s1 = _tc_mm(x, W1)
  p1 = _sc_agg[_NH1](s1, src40, dst40, ew40)
  s2 = _tc_relu_mm(p1, b1, W2)
  p2 = _sc_agg[_NH2](s2, src80, dst80, ew80)
  s3, out2 = _tc_layer3(p2, b2, W3, We, be)
  p3 = _sc_agg[_NCLASS](s3, src80, dst80, ew80)
  out1 = _tc_logsoftmax(p3, b3)
  return out1, out2


# speedup vs baseline: 14.3439x; 1.0892x over previous
import functools

import jax
import jax.numpy as jnp
from jax import lax
from jax.experimental import pallas as pl
from jax.experimental.pallas import tpu as pltpu
from jax.experimental.pallas import tpu_sc as plsc

_N = 10000
_E = 320000
_NFEAT = 128
_NH1 = 128
_NH2 = 64
_NCLASS = 16
_NSTRUC = 32

_NC = 2
_NS = 16
_NW = _NC * _NS
_EPT = _E // _NW
_PN = 10240
_RPT = _PN // _NS


def _chunk_for(D):
  return 40 if D == 128 else 125


def _make_sc_agg(D):
  nvec = D // 16
  chunk = _chunk_for(D)
  nch = _EPT // chunk
  mesh = plsc.VectorSubcoreMesh(core_axis_name="c", subcore_axis_name="s")

  @functools.partial(
      pl.kernel,
      out_type=jax.ShapeDtypeStruct((_NC, _PN, D), jnp.float32),
      mesh=mesh,
      compiler_params=pltpu.CompilerParams(use_tc_tiling_on_sc=False),
      scratch_types=[
          pltpu.VMEM((nch, chunk), jnp.int32),
          pltpu.VMEM((nch, chunk), jnp.int32),
          pltpu.VMEM((chunk,), jnp.float32),
          pltpu.VMEM((chunk,), jnp.float32),
          pltpu.VMEM((chunk, D), jnp.float32),
          pltpu.VMEM((chunk, D), jnp.float32),
          pltpu.VMEM((chunk, D), jnp.float32),
          pltpu.VMEM((chunk, D), jnp.float32),
          pltpu.VMEM_SHARED((_PN, D), jnp.float32),
          pltpu.SemaphoreType.DMA,
          pltpu.SemaphoreType.DMA,
          pltpu.SemaphoreType.DMA,
          pltpu.SemaphoreType.DMA,
          pltpu.SemaphoreType.DMA,
          pltpu.SemaphoreType.DMA,
      ],
  )
  def k(sup_hbm, src_hbm, dst_hbm, ew_hbm, out_hbm,
        src_all, dst_all, ew0, ew1, rg0, rg1, rs0, rs1, agg_sh,
        gs0, gs1, ss0, ss1, es0, es1):
    ew = (ew0, ew1)
    rg = (rg0, rg1)
    rs = (rs0, rs1)
    gs = (gs0, gs1)
    ss = (ss0, ss1)
    es = (es0, es1)
    cid = lax.axis_index("c")
    sid = lax.axis_index("s")
    wid = sid * _NC + cid
    rbase = sid * _RPT

    pltpu.sync_copy(src_hbm.at[wid], src_all)
    pltpu.sync_copy(dst_hbm.at[wid], dst_all)

    def zrow(r, carry):
      for c in range(nvec):
        rg0[r, pl.ds(c * 16, 16)] = jnp.zeros((16,), jnp.float32)
      return carry
    lax.fori_loop(0, chunk, zrow, 0)
    for j in range(_RPT // chunk):
      pltpu.sync_copy(rg0, agg_sh.at[pl.ds(rbase + j * chunk, chunk)])
    if _RPT % chunk:
      pltpu.sync_copy(
          rg0.at[pl.ds(0, _RPT % chunk)],
          agg_sh.at[pl.ds(rbase + (_RPT // chunk) * chunk, _RPT % chunk)])
    plsc.subcore_barrier()

    def issue_ew(j, b):
      pltpu.async_copy(ew_hbm.at[wid, j], ew[b], es[b])

    def wait_ew(j, b):
      pltpu.make_async_copy(ew_hbm.at[wid, j], ew[b], es[b]).wait()

    def issue_gather(j, b):
      pltpu.async_copy(sup_hbm.at[src_all.at[j]], rg[b], gs[b])

    def wait_gather(j, b):
      pltpu.make_async_copy(sup_hbm.at[src_all.at[j]], rg[b], gs[b]).wait()

    def issue_scatter(j, b):
      pltpu.async_copy(rs[b], agg_sh.at[dst_all.at[j]], ss[b], add=True)

    def wait_scatter(j, b):
      pltpu.make_async_copy(rs[b], agg_sh.at[dst_all.at[j]], ss[b]).wait()

    groups = [(g * 16, 0) for g in range(chunk // 16)]
    if chunk % 16:
      groups.append((chunk - 16, 16 - chunk % 16))

    def scale(j, b):
      for base, jj0 in groups:
        ew16 = ew[b][pl.ds(base, 16)]
        for jj in range(jj0, 16):
          w = jnp.broadcast_to(ew16[jj], (16,))
          r = base + jj
          for c in range(nvec):
            rs[b][r, pl.ds(c * 16, 16)] = rg[b][r, pl.ds(c * 16, 16)] * w

    def step(j, b):
      wait_gather(j, b)
      wait_ew(j, b)

      @pl.when(j >= 2)
      def _():
        wait_scatter(j, b)

      scale(j, b)

      @pl.when(j <= nch - 3)
      def _():
        issue_gather(j + 2, b)
        issue_ew(j + 2, b)

      issue_scatter(j, b)

    issue_ew(0, 0)
    issue_ew(1, 1)
    issue_gather(0, 0)
    issue_gather(1, 1)

    def body(g, carry):
      step(2 * g, 0)
      step(2 * g + 1, 1)
      return carry
    lax.fori_loop(0, nch // 2, body, 0)
    if nch % 2:
      step(jnp.int32(nch - 1), 0)

    wait_scatter(jnp.int32(nch - 2), (nch - 2) % 2)
    wait_scatter(jnp.int32(nch - 1), (nch - 1) % 2)

    plsc.subcore_barrier()
    pltpu.sync_copy(agg_sh.at[pl.ds(rbase, _RPT)],
                    out_hbm.at[cid, pl.ds(rbase, _RPT)])

  return k


_sc_agg = {D: _make_sc_agg(D) for D in (_NH1, _NH2, _NCLASS)}

_BR = 1000


def _tc_layer12(p, W1, b1, W2):
  def body(p_ref, w1_ref, b1_ref, w2_ref, o_ref):
    agg = jnp.dot(p_ref[0] + p_ref[1], w1_ref[...],
                  preferred_element_type=jnp.float32)
    h = jnp.maximum(agg + b1_ref[...], 0.0)
    o_ref[...] = jnp.dot(h, w2_ref[...], preferred_element_type=jnp.float32)
  return pl.pallas_call(
      body,
      grid=(_N // _BR,),
      in_specs=[pl.BlockSpec((2, _BR, _NFEAT), lambda i: (0, i, 0)),
                pl.BlockSpec((_NFEAT, _NH1), lambda i: (0, 0)),
                pl.BlockSpec((1, _NH1), lambda i: (0, 0)),
                pl.BlockSpec((_NH1, _NH2), lambda i: (0, 0))],
      out_specs=pl.BlockSpec((_BR, _NH2), lambda i: (i, 0)),
      out_shape=jax.ShapeDtypeStruct((_N, _NH2), jnp.float32),
  )(p, W1, b1.reshape(1, _NH1), W2)


def _tc_layer3(p, b2, W3, We, be):
  def body(p_ref, b2_ref, w3_ref, we_ref, be_ref, o1_ref, o2_ref):
    h = jnp.maximum(p_ref[0] + p_ref[1] + b2_ref[...], 0.0)
    o1_ref[...] = jnp.dot(h, w3_ref[...], preferred_element_type=jnp.float32)
    o2_ref[...] = jnp.dot(h, we_ref[...],
                          preferred_element_type=jnp.float32) + be_ref[...]
  return pl.pallas_call(
      body,
      grid=(_N // _BR,),
      in_specs=[pl.BlockSpec((2, _BR, _NH2), lambda i: (0, i, 0)),
                pl.BlockSpec((1, _NH2), lambda i: (0, 0)),
                pl.BlockSpec((_NH2, _NCLASS), lambda i: (0, 0)),
                pl.BlockSpec((_NH2, _NSTRUC), lambda i: (0, 0)),
                pl.BlockSpec((1, _NSTRUC), lambda i: (0, 0))],
      out_specs=[pl.BlockSpec((_BR, _NCLASS), lambda i: (i, 0)),
                 pl.BlockSpec((_BR, _NSTRUC), lambda i: (i, 0))],
      out_shape=[jax.ShapeDtypeStruct((_N, _NCLASS), jnp.float32),
                 jax.ShapeDtypeStruct((_N, _NSTRUC), jnp.float32)],
  )(p, b2.reshape(1, _NH2), W3, We, be.reshape(1, _NSTRUC))


def _tc_logsoftmax(p, b):
  def body(p_ref, b_ref, o_ref):
    o = p_ref[0] + p_ref[1] + b_ref[...]
    s = o - jnp.max(o, axis=1, keepdims=True)
    o_ref[...] = s - jnp.log(jnp.sum(jnp.exp(s), axis=1, keepdims=True))
  return pl.pallas_call(
      body,
      grid=(_N // _BR,),
      in_specs=[pl.BlockSpec((2, _BR, _NCLASS), lambda i: (0, i, 0)),
                pl.BlockSpec((1, _NCLASS), lambda i: (0, 0))],
      out_specs=pl.BlockSpec((_BR, _NCLASS), lambda i: (i, 0)),
      out_shape=jax.ShapeDtypeStruct((_N, _NCLASS), jnp.float32),
  )(p, b.reshape(1, _NCLASS))


def _edges_for(edge_index, edge_weight, D):
  chunk = _chunk_for(D)
  nch = _EPT // chunk
  return (edge_index[0].reshape(_NW, nch, chunk),
          edge_index[1].reshape(_NW, nch, chunk),
          edge_weight.reshape(_NW, nch, chunk))


def kernel(x, edge_index, edge_weight, W1, b1, W2, b2, W3, b3, We, be):
  src40, dst40, ew40 = _edges_for(edge_index, edge_weight, _NH1)
  src80, dst80, ew80 = _edges_for(edge_index, edge_weight, _NH2)
  p1 = _sc_agg[_NFEAT](x, src40, dst40, ew40)
  s2 = _tc_layer12(p1, W1, b1, W2)
  p2 = _sc_agg[_NH2](s2, src80, dst80, ew80)
  s3, out2 = _tc_layer3(p2, b2, W3, We, be)
  p3 = _sc_agg[_NCLASS](s3, src80, dst80, ew80)
  out1 = _tc_logsoftmax(p3, b3)
  return out1, out2


# speedup vs baseline: 15.3181x; 1.0679x over previous
import functools

import jax
import jax.numpy as jnp
from jax import lax
from jax.experimental import pallas as pl
from jax.experimental.pallas import tpu as pltpu
from jax.experimental.pallas import tpu_sc as plsc

_N = 10000
_E = 320000
_NFEAT = 128
_NH1 = 128
_NH2 = 64
_NCLASS = 16
_NSTRUC = 32

_NC = 2
_NS = 16
_NW = _NC * _NS
_EPT = _E // _NW
_PN = 10240
_RPT = _PN // _NS


def _chunk_for(D):
  return 80 if D == 128 else 125


def _make_sc_agg(D):
  nvec = D // 16
  chunk = _chunk_for(D)
  nch = _EPT // chunk
  stream_idx = D == 128
  mesh = plsc.VectorSubcoreMesh(core_axis_name="c", subcore_axis_name="s")

  if stream_idx:
    idx_scratch = [pltpu.VMEM((chunk,), jnp.int32)] * 4
    idx_sems = [pltpu.SemaphoreType.DMA] * 4
  else:
    idx_scratch = [pltpu.VMEM((nch, chunk), jnp.int32)] * 2
    idx_sems = []

  @functools.partial(
      pl.kernel,
      out_type=jax.ShapeDtypeStruct((_NC, _PN, D), jnp.float32),
      mesh=mesh,
      compiler_params=pltpu.CompilerParams(use_tc_tiling_on_sc=False),
      scratch_types=idx_scratch + [
          pltpu.VMEM((chunk,), jnp.float32),
          pltpu.VMEM((chunk,), jnp.float32),
          pltpu.VMEM((chunk, D), jnp.float32),
          pltpu.VMEM((chunk, D), jnp.float32),
          pltpu.VMEM((chunk, D), jnp.float32),
          pltpu.VMEM((chunk, D), jnp.float32),
          pltpu.VMEM_SHARED((_PN, D), jnp.float32),
          pltpu.SemaphoreType.DMA,
          pltpu.SemaphoreType.DMA,
          pltpu.SemaphoreType.DMA,
          pltpu.SemaphoreType.DMA,
          pltpu.SemaphoreType.DMA,
          pltpu.SemaphoreType.DMA,
      ] + idx_sems,
  )
  def k(sup_hbm, src_hbm, dst_hbm, ew_hbm, out_hbm, *scr):
    if stream_idx:
      (src_v0, src_v1, dst_v0, dst_v1, ew0, ew1, rg0, rg1, rs0, rs1,
       agg_sh, gs0, gs1, ss0, ss1, es0, es1, is0, is1, js0, js1) = scr
      src_v = (src_v0, src_v1)
      dst_v = (dst_v0, dst_v1)
      isem = (is0, is1)
      jsem = (js0, js1)
    else:
      (src_all, dst_all, ew0, ew1, rg0, rg1, rs0, rs1,
       agg_sh, gs0, gs1, ss0, ss1, es0, es1) = scr
    ew = (ew0, ew1)
    rg = (rg0, rg1)
    rs = (rs0, rs1)
    gs = (gs0, gs1)
    ss = (ss0, ss1)
    es = (es0, es1)
    cid = lax.axis_index("c")
    sid = lax.axis_index("s")
    wid = sid * _NC + cid
    rbase = sid * _RPT

    if not stream_idx:
      pltpu.sync_copy(src_hbm.at[wid], src_all)
      pltpu.sync_copy(dst_hbm.at[wid], dst_all)

    def zrow(r, carry):
      for c in range(nvec):
        rg0[r, pl.ds(c * 16, 16)] = jnp.zeros((16,), jnp.float32)
      return carry
    lax.fori_loop(0, chunk, zrow, 0)
    for j in range(_RPT // chunk):
      pltpu.sync_copy(rg0, agg_sh.at[pl.ds(rbase + j * chunk, chunk)])
    if _RPT % chunk:
      pltpu.sync_copy(
          rg0.at[pl.ds(0, _RPT % chunk)],
          agg_sh.at[pl.ds(rbase + (_RPT // chunk) * chunk, _RPT % chunk)])
    plsc.subcore_barrier()

    def issue_ew(j, b):
      pltpu.async_copy(ew_hbm.at[wid, j], ew[b], es[b])

    def wait_ew(j, b):
      pltpu.make_async_copy(ew_hbm.at[wid, j], ew[b], es[b]).wait()

    if stream_idx:
      def issue_src(j, b):
        pltpu.async_copy(src_hbm.at[wid, j], src_v[b], isem[b])

      def wait_src(j, b):
        pltpu.make_async_copy(src_hbm.at[wid, j], src_v[b], isem[b]).wait()

      def issue_dst(j, b):
        pltpu.async_copy(dst_hbm.at[wid, j], dst_v[b], jsem[b])

      def wait_dst(j, b):
        pltpu.make_async_copy(dst_hbm.at[wid, j], dst_v[b], jsem[b]).wait()

      def gather_src(j, b):
        return sup_hbm.at[src_v[b]]

      def scatter_dst(j, b):
        return agg_sh.at[dst_v[b]]
    else:
      def gather_src(j, b):
        return sup_hbm.at[src_all.at[j]]

      def scatter_dst(j, b):
        return agg_sh.at[dst_all.at[j]]

    def issue_gather(j, b):
      pltpu.async_copy(gather_src(j, b), rg[b], gs[b])

    def wait_gather(j, b):
      pltpu.make_async_copy(gather_src(j, b), rg[b], gs[b]).wait()

    def issue_scatter(j, b):
      pltpu.async_copy(rs[b], scatter_dst(j, b), ss[b], add=True)

    def wait_scatter(j, b):
      pltpu.make_async_copy(rs[b], scatter_dst(j, b), ss[b]).wait()

    groups = [(g * 16, 0) for g in range(chunk // 16)]
    if chunk % 16:
      groups.append((chunk - 16, 16 - chunk % 16))

    def scale(j, b):
      for base, jj0 in groups:
        ew16 = ew[b][pl.ds(base, 16)]
        for jj in range(jj0, 16):
          w = jnp.broadcast_to(ew16[jj], (16,))
          r = base + jj
          for c in range(nvec):
            rs[b][r, pl.ds(c * 16, 16)] = rg[b][r, pl.ds(c * 16, 16)] * w

    if stream_idx:
      def step(j, b):
        wait_gather(j, b)
        wait_ew(j, b)

        @pl.when(j >= 2)
        def _():
          wait_scatter(j, b)

        @pl.when(j <= nch - 3)
        def _():
          issue_src(j + 2, b)

        issue_dst(j, b)

        scale(j, b)

        @pl.when(j <= nch - 3)
        def _():
          issue_ew(j + 2, b)
          wait_src(j + 2, b)
          issue_gather(j + 2, b)

        wait_dst(j, b)
        issue_scatter(j, b)

      issue_ew(0, 0)
      issue_ew(1, 1)
      issue_src(0, 0)
      issue_src(1, 1)
      wait_src(0, 0)
      issue_gather(0, 0)
      wait_src(1, 1)
      issue_gather(1, 1)
    else:
      def step(j, b):
        wait_gather(j, b)
        wait_ew(j, b)

        @pl.when(j >= 2)
        def _():
          wait_scatter(j, b)

        scale(j, b)

        @pl.when(j <= nch - 3)
        def _():
          issue_gather(j + 2, b)
          issue_ew(j + 2, b)

        issue_scatter(j, b)

      issue_ew(0, 0)
      issue_ew(1, 1)
      issue_gather(0, 0)
      issue_gather(1, 1)

    def body(g, carry):
      step(2 * g, 0)
      step(2 * g + 1, 1)
      return carry
    lax.fori_loop(0, nch // 2, body, 0)
    if nch % 2:
      step(jnp.int32(nch - 1), 0)

    wait_scatter(jnp.int32(nch - 2), (nch - 2) % 2)
    wait_scatter(jnp.int32(nch - 1), (nch - 1) % 2)

    plsc.subcore_barrier()
    pltpu.sync_copy(agg_sh.at[pl.ds(rbase, _RPT)],
                    out_hbm.at[cid, pl.ds(rbase, _RPT)])

  return k


_sc_agg = {D: _make_sc_agg(D) for D in (_NH1, _NH2, _NCLASS)}

_BR = 1000


def _tc_layer12(p, W1, b1, W2):
  def body(p_ref, w1_ref, b1_ref, w2_ref, o_ref):
    agg = jnp.dot(p_ref[0] + p_ref[1], w1_ref[...],
                  preferred_element_type=jnp.float32)
    h = jnp.maximum(agg + b1_ref[...], 0.0)
    o_ref[...] = jnp.dot(h, w2_ref[...], preferred_element_type=jnp.float32)
  return pl.pallas_call(
      body,
      grid=(_N // _BR,),
      in_specs=[pl.BlockSpec((2, _BR, _NFEAT), lambda i: (0, i, 0)),
                pl.BlockSpec((_NFEAT, _NH1), lambda i: (0, 0)),
                pl.BlockSpec((1, _NH1), lambda i: (0, 0)),
                pl.BlockSpec((_NH1, _NH2), lambda i: (0, 0))],
      out_specs=pl.BlockSpec((_BR, _NH2), lambda i: (i, 0)),
      out_shape=jax.ShapeDtypeStruct((_N, _NH2), jnp.float32),
  )(p, W1, b1.reshape(1, _NH1), W2)


def _tc_layer3(p, b2, W3, We, be):
  def body(p_ref, b2_ref, w3_ref, we_ref, be_ref, o1_ref, o2_ref):
    h = jnp.maximum(p_ref[0] + p_ref[1] + b2_ref[...], 0.0)
    o1_ref[...] = jnp.dot(h, w3_ref[...], preferred_element_type=jnp.float32)
    o2_ref[...] = jnp.dot(h, we_ref[...],
                          preferred_element_type=jnp.float32) + be_ref[...]
  return pl.pallas_call(
      body,
      grid=(_N // _BR,),
      in_specs=[pl.BlockSpec((2, _BR, _NH2), lambda i: (0, i, 0)),
                pl.BlockSpec((1, _NH2), lambda i: (0, 0)),
                pl.BlockSpec((_NH2, _NCLASS), lambda i: (0, 0)),
                pl.BlockSpec((_NH2, _NSTRUC), lambda i: (0, 0)),
                pl.BlockSpec((1, _NSTRUC), lambda i: (0, 0))],
      out_specs=[pl.BlockSpec((_BR, _NCLASS), lambda i: (i, 0)),
                 pl.BlockSpec((_BR, _NSTRUC), lambda i: (i, 0))],
      out_shape=[jax.ShapeDtypeStruct((_N, _NCLASS), jnp.float32),
                 jax.ShapeDtypeStruct((_N, _NSTRUC), jnp.float32)],
  )(p, b2.reshape(1, _NH2), W3, We, be.reshape(1, _NSTRUC))


def _tc_logsoftmax(p, b):
  def body(p_ref, b_ref, o_ref):
    o = p_ref[0] + p_ref[1] + b_ref[...]
    s = o - jnp.max(o, axis=1, keepdims=True)
    o_ref[...] = s - jnp.log(jnp.sum(jnp.exp(s), axis=1, keepdims=True))
  return pl.pallas_call(
      body,
      grid=(_N // _BR,),
      in_specs=[pl.BlockSpec((2, _BR, _NCLASS), lambda i: (0, i, 0)),
                pl.BlockSpec((1, _NCLASS), lambda i: (0, 0))],
      out_specs=pl.BlockSpec((_BR, _NCLASS), lambda i: (i, 0)),
      out_shape=jax.ShapeDtypeStruct((_N, _NCLASS), jnp.float32),
  )(p, b.reshape(1, _NCLASS))


def _edges_for(edge_index, edge_weight, D):
  chunk = _chunk_for(D)
  nch = _EPT // chunk
  return (edge_index[0].reshape(_NW, nch, chunk),
          edge_index[1].reshape(_NW, nch, chunk),
          edge_weight.reshape(_NW, nch, chunk))


def kernel(x, edge_index, edge_weight, W1, b1, W2, b2, W3, b3, We, be):
  src40, dst40, ew40 = _edges_for(edge_index, edge_weight, _NH1)
  src80, dst80, ew80 = _edges_for(edge_index, edge_weight, _NH2)
  p1 = _sc_agg[_NFEAT](x, src40, dst40, ew40)
  s2 = _tc_layer12(p1, W1, b1, W2)
  p2 = _sc_agg[_NH2](s2, src80, dst80, ew80)
  s3, out2 = _tc_layer3(p2, b2, W3, We, be)
  p3 = _sc_agg[_NCLASS](s3, src80, dst80, ew80)
  out1 = _tc_logsoftmax(p3, b3)
  return out1, out2
